# baseline (speedup 1.0000x reference)
import functools

import jax
import jax.numpy as jnp
from jax import lax
from jax.experimental import pallas as pl
from jax.experimental.pallas import tpu as pltpu
from jax.experimental.pallas import tpu_sc as plsc

Z = 128
R = 64
NCOMBO = 4


def _softplus(x):
    return jnp.maximum(x, 0.0) + jnp.log1p(jnp.exp(-jnp.abs(x)))


def _table_kernel(mu_c_ref, lr_c_ref, d_c_ref, mu_s_ref, lr_s_ref, d_s_ref,
                  ctable_ref, mu_t_ref):
    row = jax.lax.broadcasted_iota(jnp.int32, (Z, Z), 0)
    col = jax.lax.broadcasted_iota(jnp.int32, (Z, Z), 1)
    diag_mask = (row == col).astype(jnp.float32)
    for e in range(2):
        lrc = lr_c_ref[e]
        cc = jax.lax.dot_general(lrc, lrc, (((1,), (1,)), ((), ())),
                                 preferred_element_type=jnp.float32)
        dc = _softplus(d_c_ref[e]) + 1e-6
        ctable_ref[e] = cc + diag_mask * dc[None, :]
    for combo in range(NCOMBO):
        e = combo % 2
        lrs = lr_s_ref[combo]
        cs = jax.lax.dot_general(lrs, lrs, (((1,), (1,)), ((), ())),
                                 preferred_element_type=jnp.float32)
        ds = _softplus(d_s_ref[combo]) + 1e-6
        ctable_ref[2 + combo] = cs + diag_mask * ds[None, :]
        mu_t_ref[combo, 0:Z] = mu_c_ref[e, :]
        mu_t_ref[combo, Z:2 * Z] = mu_s_ref[combo, :]


def _build_tables(mu_causal, low_rank_causal, diag_causal,
                  mu_spurious, low_rank_spurious, diag_spurious):
    mu_s = mu_spurious.reshape(NCOMBO, Z)
    lr_s = low_rank_spurious.reshape(NCOMBO, Z, R)
    d_s = diag_spurious.reshape(NCOMBO, Z)
    return pl.pallas_call(
        _table_kernel,
        out_shape=(
            jax.ShapeDtypeStruct((6, Z, Z), jnp.float32),
            jax.ShapeDtypeStruct((NCOMBO, 2 * Z), jnp.float32),
        ),
    )(mu_causal, low_rank_causal, diag_causal, mu_s, lr_s, d_s)


def _sc_gather(b):
    info = plsc.get_sparse_core_info()
    nc, ns = info.num_cores, info.num_subcores
    nw = nc * ns
    bpw = b // nw
    mesh = plsc.VectorSubcoreMesh(core_axis_name="c", subcore_axis_name="s")

    @functools.partial(
        pl.kernel,
        mesh=mesh,
        out_type=(
            jax.ShapeDtypeStruct((b, 2 * Z, 2 * Z), jnp.float32),
            jax.ShapeDtypeStruct((b, 2 * Z), jnp.float32),
        ),
        scratch_types=[
            pltpu.VMEM((bpw,), jnp.int32),
            pltpu.VMEM((bpw,), jnp.int32),
            pltpu.VMEM((bpw,), jnp.int32),
            pltpu.VMEM((bpw, 2 * Z), jnp.float32),
            pltpu.VMEM((2, 64, 2 * Z), jnp.float32),
            pltpu.VMEM((2, 64, 2 * Z), jnp.float32),
            pltpu.SemaphoreType.DMA,
            pltpu.SemaphoreType.DMA,
            pltpu.SemaphoreType.DMA,
            pltpu.SemaphoreType.DMA,
            pltpu.SemaphoreType.DMA,
            pltpu.SemaphoreType.DMA,
            pltpu.SemaphoreType.DMA,
            pltpu.SemaphoreType.DMA,
            pltpu.SemaphoreType.DMA,
            pltpu.SemaphoreType.DMA,
        ],
    )
    def k(ctable, cidx, sidx, combo, mu_t, zq,
          cov_out, mu_out,
          cidx_v, sidx_v, combo_v, mu_rows, buft, bufb,
          gt0, gt1, gb0, gb1, st0, st1, sb0, sb1, zsem, msem):
        gt = [gt0, gt1]
        gb = [gb0, gb1]
        st = [st0, st1]
        sb = [sb0, sb1]
        wid = lax.axis_index("s") * nc + lax.axis_index("c")
        base = wid * bpw
        pltpu.sync_copy(cidx.at[pl.ds(base, bpw)], cidx_v)
        pltpu.sync_copy(sidx.at[pl.ds(base, bpw)], sidx_v)
        pltpu.sync_copy(combo.at[pl.ds(base, bpw)], combo_v)
        mu_cp = pltpu.async_copy(mu_t.at[combo_v], mu_rows, msem)
        z0 = pltpu.async_copy(zq, buft.at[0, :, pl.ds(Z, Z)], zsem)
        z1 = pltpu.async_copy(zq, buft.at[1, :, pl.ds(Z, Z)], zsem)
        z2 = pltpu.async_copy(zq, bufb.at[0, :, pl.ds(0, Z)], zsem)
        z3 = pltpu.async_copy(zq, bufb.at[1, :, pl.ds(0, Z)], zsem)
        z0.wait()
        z1.wait()
        z2.wait()
        z3.wait()
        pend_t = [None, None]
        pend_b = [None, None]
        cvecs = [cidx_v[pl.ds(16 * t, 16)] for t in range(bpw // 16)]
        svecs = [sidx_v[pl.ds(16 * t, 16)] for t in range(bpw // 16)]
        for j in range(bpw):
            c = cvecs[j // 16][j % 16]
            s = svecs[j // 16][j % 16]
            i_abs = base + j
            for h in range(2):
                slot = (j * 2 + h) % 2
                if pend_t[slot] is not None:
                    pend_t[slot].wait()
                if pend_b[slot] is not None:
                    pend_b[slot].wait()
                g1 = pltpu.async_copy(
                    ctable.at[c, pl.ds(64 * h, 64), :],
                    buft.at[slot, :, pl.ds(0, Z)], gt[slot])
                g2 = pltpu.async_copy(
                    ctable.at[s, pl.ds(64 * h, 64), :],
                    bufb.at[slot, :, pl.ds(Z, Z)], gb[slot])
                g1.wait()
                g2.wait()
                pend_t[slot] = pltpu.async_copy(
                    buft.at[slot],
                    cov_out.at[i_abs, pl.ds(64 * h, 64), :], st[slot])
                pend_b[slot] = pltpu.async_copy(
                    bufb.at[slot],
                    cov_out.at[i_abs, pl.ds(Z + 64 * h, 64), :], sb[slot])
        for d in pend_t + pend_b:
            if d is not None:
                d.wait()
        mu_cp.wait()
        pltpu.sync_copy(mu_rows, mu_out.at[pl.ds(base, bpw)])

    return k


def kernel(y, e, mu_causal, low_rank_causal, diag_causal,
           mu_spurious, low_rank_spurious, diag_spurious):
    yi = y.astype(jnp.int32)
    ei = e.astype(jnp.int32)
    combo = yi * 2 + ei
    cidx = ei
    sidx = combo + 2
    ctable, mu_t = _build_tables(mu_causal, low_rank_causal, diag_causal,
                                 mu_spurious, low_rank_spurious, diag_spurious)
    b = y.shape[0]
    zq = jnp.zeros((64, Z), dtype=jnp.float32)
    cov, mu = _sc_gather(b)(ctable, cidx, sidx, combo, mu_t, zq)
    return (mu, cov)


# speedup vs baseline: 1.0265x; 1.0265x over previous
import functools

import jax
import jax.numpy as jnp
from jax import lax
from jax.experimental import pallas as pl
from jax.experimental.pallas import tpu as pltpu
from jax.experimental.pallas import tpu_sc as plsc

Z = 128
R = 64
NCOMBO = 4


def _softplus(x):
    return jnp.maximum(x, 0.0) + jnp.log1p(jnp.exp(-jnp.abs(x)))


def _table_kernel(mu_c_ref, lr_c_ref, d_c_ref, mu_s_ref, lr_s_ref, d_s_ref,
                  ctable_ref, mu_t_ref):
    row = jax.lax.broadcasted_iota(jnp.int32, (Z, Z), 0)
    col = jax.lax.broadcasted_iota(jnp.int32, (Z, Z), 1)
    diag_mask = (row == col).astype(jnp.float32)
    for e in range(2):
        lrc = lr_c_ref[e]
        cc = jax.lax.dot_general(lrc, lrc, (((1,), (1,)), ((), ())),
                                 preferred_element_type=jnp.float32)
        dc = _softplus(d_c_ref[e]) + 1e-6
        ctable_ref[e] = cc + diag_mask * dc[None, :]
    for combo in range(NCOMBO):
        e = combo % 2
        lrs = lr_s_ref[combo]
        cs = jax.lax.dot_general(lrs, lrs, (((1,), (1,)), ((), ())),
                                 preferred_element_type=jnp.float32)
        ds = _softplus(d_s_ref[combo]) + 1e-6
        ctable_ref[2 + combo] = cs + diag_mask * ds[None, :]
        mu_t_ref[combo, 0:Z] = mu_c_ref[e, :]
        mu_t_ref[combo, Z:2 * Z] = mu_s_ref[combo, :]


def _build_tables(mu_causal, low_rank_causal, diag_causal,
                  mu_spurious, low_rank_spurious, diag_spurious):
    mu_s = mu_spurious.reshape(NCOMBO, Z)
    lr_s = low_rank_spurious.reshape(NCOMBO, Z, R)
    d_s = diag_spurious.reshape(NCOMBO, Z)
    return pl.pallas_call(
        _table_kernel,
        out_shape=(
            jax.ShapeDtypeStruct((6, Z, Z), jnp.float32),
            jax.ShapeDtypeStruct((NCOMBO, 2 * Z), jnp.float32),
        ),
    )(mu_causal, low_rank_causal, diag_causal, mu_s, lr_s, d_s)


def _sc_gather(b):
    info = plsc.get_sparse_core_info()
    nc, ns = info.num_cores, info.num_subcores
    nw = nc * ns
    bpw = b // nw
    mesh = plsc.VectorSubcoreMesh(core_axis_name="c", subcore_axis_name="s")

    @functools.partial(
        pl.kernel,
        mesh=mesh,
        out_type=(
            jax.ShapeDtypeStruct((b, 2 * Z, 2 * Z), jnp.float32),
            jax.ShapeDtypeStruct((b, 2 * Z), jnp.float32),
        ),
        scratch_types=[
            pltpu.VMEM((bpw,), jnp.int32),
            pltpu.VMEM((bpw,), jnp.int32),
            pltpu.VMEM((bpw,), jnp.int32),
            pltpu.VMEM((bpw, 2 * Z), jnp.float32),
            pltpu.VMEM((3, Z, Z), jnp.float32),
            pltpu.VMEM((3, Z, Z), jnp.float32),
            pltpu.VMEM((Z, Z), jnp.float32),
            pltpu.SemaphoreType.DMA,
            pltpu.SemaphoreType.DMA,
            pltpu.SemaphoreType.DMA,
            pltpu.SemaphoreType.DMA,
            pltpu.SemaphoreType.DMA,
            pltpu.SemaphoreType.DMA,
            pltpu.SemaphoreType.DMA,
            pltpu.SemaphoreType.DMA,
        ],
    )
    def k(ctable, cidx, sidx, combo, mu_t, zq,
          cov_out, mu_out,
          cidx_v, sidx_v, combo_v, mu_rows, bufc, bufs, zq_v,
          g0, g1, g2, s0, s1, s2, zsem, msem):
        S = 3
        gsem = [g0, g1, g2]
        ssem = [s0, s1, s2]
        wid = lax.axis_index("s") * nc + lax.axis_index("c")
        base = wid * bpw
        pltpu.sync_copy(cidx.at[pl.ds(base, bpw)], cidx_v)
        pltpu.sync_copy(sidx.at[pl.ds(base, bpw)], sidx_v)
        pltpu.sync_copy(combo.at[pl.ds(base, bpw)], combo_v)
        mu_cp = pltpu.async_copy(mu_t.at[combo_v], mu_rows, msem)
        pltpu.sync_copy(zq, zq_v)
        cvecs = [cidx_v[pl.ds(16 * t, 16)] for t in range(bpw // 16)]
        svecs = [sidx_v[pl.ds(16 * t, 16)] for t in range(bpw // 16)]

        def gidx(j):
            return cvecs[j // 16][j % 16], svecs[j // 16][j % 16]

        def start_gather(j):
            c, s = gidx(j)
            slot = j % S
            d1 = pltpu.async_copy(ctable.at[c], bufc.at[slot], gsem[slot])
            d2 = pltpu.async_copy(ctable.at[s], bufs.at[slot], gsem[slot])
            return (d1, d2)

        pend_g = [None] * S
        pend_s = [None] * S
        pend_z = []
        for j in range(min(S, bpw)):
            pend_g[j % S] = start_gather(j)
        for j in range(bpw):
            slot = j % S
            i_abs = base + j
            pend_z.append(pltpu.async_copy(
                zq_v, cov_out.at[i_abs, pl.ds(0, Z), pl.ds(Z, Z)], zsem))
            pend_z.append(pltpu.async_copy(
                zq_v, cov_out.at[i_abs, pl.ds(Z, Z), pl.ds(0, Z)], zsem))
            pend_g[slot][0].wait()
            pend_g[slot][1].wait()
            d1 = pltpu.async_copy(
                bufc.at[slot], cov_out.at[i_abs, pl.ds(0, Z), pl.ds(0, Z)],
                ssem[slot])
            d2 = pltpu.async_copy(
                bufs.at[slot], cov_out.at[i_abs, pl.ds(Z, Z), pl.ds(Z, Z)],
                ssem[slot])
            pend_s[slot] = (d1, d2)
            if j + S < bpw:
                pend_s[slot][0].wait()
                pend_s[slot][1].wait()
                pend_s[slot] = None
                pend_g[slot] = start_gather(j + S)
            if len(pend_z) > 8:
                pend_z.pop(0).wait()
                pend_z.pop(0).wait()
        for p in pend_s:
            if p is not None:
                p[0].wait()
                p[1].wait()
        for d in pend_z:
            d.wait()
        mu_cp.wait()
        pltpu.sync_copy(mu_rows, mu_out.at[pl.ds(base, bpw)])

    return k


def kernel(y, e, mu_causal, low_rank_causal, diag_causal,
           mu_spurious, low_rank_spurious, diag_spurious):
    yi = y.astype(jnp.int32)
    ei = e.astype(jnp.int32)
    combo = yi * 2 + ei
    cidx = ei
    sidx = combo + 2
    ctable, mu_t = _build_tables(mu_causal, low_rank_causal, diag_causal,
                                 mu_spurious, low_rank_spurious, diag_spurious)
    b = y.shape[0]
    zq = jnp.zeros((Z, Z), dtype=jnp.float32)
    cov, mu = _sc_gather(b)(ctable, cidx, sidx, combo, mu_t, zq)
    return (mu, cov)


# speedup vs baseline: 2.0524x; 1.9994x over previous
import functools

import jax
import jax.numpy as jnp
from jax import lax
from jax.experimental import pallas as pl
from jax.experimental.pallas import tpu as pltpu
from jax.experimental.pallas import tpu_sc as plsc

Z = 128
R = 64
NCOMBO = 4
BB = 8


def _softplus(x):
    return jnp.maximum(x, 0.0) + jnp.log1p(jnp.exp(-jnp.abs(x)))


def _table_kernel(mu_c_ref, lr_c_ref, d_c_ref, mu_s_ref, lr_s_ref, d_s_ref,
                  cov_t_ref, mu_t_ref):
    row = jax.lax.broadcasted_iota(jnp.int32, (Z, Z), 0)
    col = jax.lax.broadcasted_iota(jnp.int32, (Z, Z), 1)
    diag_mask = (row == col).astype(jnp.float32)
    zeros_blk = jnp.zeros((Z, Z), dtype=jnp.float32)
    for combo in range(NCOMBO):
        e = combo % 2
        lrc = lr_c_ref[e]
        cc = jax.lax.dot_general(lrc, lrc, (((1,), (1,)), ((), ())),
                                 preferred_element_type=jnp.float32)
        dc = _softplus(d_c_ref[e]) + 1e-6
        cc = cc + diag_mask * dc[None, :]
        lrs = lr_s_ref[combo]
        cs = jax.lax.dot_general(lrs, lrs, (((1,), (1,)), ((), ())),
                                 preferred_element_type=jnp.float32)
        ds = _softplus(d_s_ref[combo]) + 1e-6
        cs = cs + diag_mask * ds[None, :]
        cov_t_ref[combo, 0:Z, 0:Z] = cc
        cov_t_ref[combo, 0:Z, Z:2 * Z] = zeros_blk
        cov_t_ref[combo, Z:2 * Z, 0:Z] = zeros_blk
        cov_t_ref[combo, Z:2 * Z, Z:2 * Z] = cs
        mu_t_ref[combo, 0:Z] = mu_c_ref[e, :]
        mu_t_ref[combo, Z:2 * Z] = mu_s_ref[combo, :]


def _build_tables(mu_causal, low_rank_causal, diag_causal,
                  mu_spurious, low_rank_spurious, diag_spurious):
    mu_s = mu_spurious.reshape(NCOMBO, Z)
    lr_s = low_rank_spurious.reshape(NCOMBO, Z, R)
    d_s = diag_spurious.reshape(NCOMBO, Z)
    return pl.pallas_call(
        _table_kernel,
        out_shape=(
            jax.ShapeDtypeStruct((NCOMBO, 2 * Z, 2 * Z), jnp.float32),
            jax.ShapeDtypeStruct((NCOMBO, 2 * Z), jnp.float32),
        ),
    )(mu_causal, low_rank_causal, diag_causal, mu_s, lr_s, d_s)


def _cov_kernel(combo_ref, cov_t_ref, cov_out_ref):
    i = pl.program_id(0)
    for j in range(BB):
        c = combo_ref[i * BB + j]
        cov_out_ref[j] = cov_t_ref[c]


def _assemble_cov(combo, cov_t, b):
    return pl.pallas_call(
        _cov_kernel,
        grid_spec=pltpu.PrefetchScalarGridSpec(
            num_scalar_prefetch=1,
            grid=(b // BB,),
            in_specs=[
                pl.BlockSpec((NCOMBO, 2 * Z, 2 * Z), lambda i, c: (0, 0, 0)),
            ],
            out_specs=pl.BlockSpec((BB, 2 * Z, 2 * Z), lambda i, c: (i, 0, 0)),
        ),
        out_shape=jax.ShapeDtypeStruct((b, 2 * Z, 2 * Z), jnp.float32),
    )(combo, cov_t)


def _sc_mu_gather(b):
    info = plsc.get_sparse_core_info()
    nc, ns = info.num_cores, info.num_subcores
    nw = nc * ns
    bpw = b // nw
    mesh = plsc.VectorSubcoreMesh(core_axis_name="c", subcore_axis_name="s")

    @functools.partial(
        pl.kernel,
        mesh=mesh,
        out_type=jax.ShapeDtypeStruct((b, 2 * Z), jnp.float32),
        scratch_types=[
            pltpu.VMEM((bpw,), jnp.int32),
            pltpu.VMEM((bpw, 2 * Z), jnp.float32),
            pltpu.SemaphoreType.DMA,
        ],
    )
    def k(mu_t, combo, mu_out, combo_v, rows_v, sem):
        wid = lax.axis_index("s") * nc + lax.axis_index("c")
        base = wid * bpw
        pltpu.sync_copy(combo.at[pl.ds(base, bpw)], combo_v)
        pltpu.async_copy(mu_t.at[combo_v], rows_v, sem).wait()
        pltpu.sync_copy(rows_v, mu_out.at[pl.ds(base, bpw)])

    return k


def kernel(y, e, mu_causal, low_rank_causal, diag_causal,
           mu_spurious, low_rank_spurious, diag_spurious):
    combo = (y.astype(jnp.int32) * 2 + e.astype(jnp.int32))
    cov_t, mu_t = _build_tables(mu_causal, low_rank_causal, diag_causal,
                                mu_spurious, low_rank_spurious, diag_spurious)
    b = y.shape[0]
    cov = _assemble_cov(combo, cov_t, b)
    mu = _sc_mu_gather(b)(mu_t, combo)
    return (mu, cov)


# speedup vs baseline: 2.0532x; 1.0004x over previous
import functools

import jax
import jax.numpy as jnp
from jax import lax
from jax.experimental import pallas as pl
from jax.experimental.pallas import tpu as pltpu
from jax.experimental.pallas import tpu_sc as plsc

Z = 128
R = 64
NCOMBO = 4
BB = 8


def _softplus(x):
    return jnp.maximum(x, 0.0) + jnp.log1p(jnp.exp(-jnp.abs(x)))


def _table_kernel(mu_c_ref, lr_c_ref, d_c_ref, mu_s_ref, lr_s_ref, d_s_ref,
                  cov_t_ref, mu_t_ref):
    row = jax.lax.broadcasted_iota(jnp.int32, (Z, Z), 0)
    col = jax.lax.broadcasted_iota(jnp.int32, (Z, Z), 1)
    diag_mask = (row == col).astype(jnp.float32)
    zeros_blk = jnp.zeros((Z, Z), dtype=jnp.float32)
    for combo in range(NCOMBO):
        e = combo % 2
        lrc = lr_c_ref[e]
        cc = jax.lax.dot_general(lrc, lrc, (((1,), (1,)), ((), ())),
                                 preferred_element_type=jnp.float32)
        dc = _softplus(d_c_ref[e]) + 1e-6
        cc = cc + diag_mask * dc[None, :]
        lrs = lr_s_ref[combo]
        cs = jax.lax.dot_general(lrs, lrs, (((1,), (1,)), ((), ())),
                                 preferred_element_type=jnp.float32)
        ds = _softplus(d_s_ref[combo]) + 1e-6
        cs = cs + diag_mask * ds[None, :]
        cov_t_ref[combo, 0:Z, 0:Z] = cc
        cov_t_ref[combo, 0:Z, Z:2 * Z] = zeros_blk
        cov_t_ref[combo, Z:2 * Z, 0:Z] = zeros_blk
        cov_t_ref[combo, Z:2 * Z, Z:2 * Z] = cs
        mu_t_ref[combo, 0:Z] = mu_c_ref[e, :]
        mu_t_ref[combo, Z:2 * Z] = mu_s_ref[combo, :]


def _build_tables(mu_causal, low_rank_causal, diag_causal,
                  mu_spurious, low_rank_spurious, diag_spurious):
    mu_s = mu_spurious.reshape(NCOMBO, Z)
    lr_s = low_rank_spurious.reshape(NCOMBO, Z, R)
    d_s = diag_spurious.reshape(NCOMBO, Z)
    return pl.pallas_call(
        _table_kernel,
        out_shape=(
            jax.ShapeDtypeStruct((NCOMBO, 2 * Z, 2 * Z), jnp.float32),
            jax.ShapeDtypeStruct((NCOMBO, 2 * Z), jnp.float32),
        ),
    )(mu_causal, low_rank_causal, diag_causal, mu_s, lr_s, d_s)


def _cov_kernel(combo_ref, cov_t_ref, cov_out_ref):
    i = pl.program_id(0)
    for j in range(BB):
        c = combo_ref[i * BB + j]
        cov_out_ref[j] = cov_t_ref[c]


def _assemble_cov(combo, cov_t, b):
    return pl.pallas_call(
        _cov_kernel,
        grid_spec=pltpu.PrefetchScalarGridSpec(
            num_scalar_prefetch=1,
            grid=(b // BB,),
            in_specs=[
                pl.BlockSpec((NCOMBO, 2 * Z, 2 * Z), lambda i, c: (0, 0, 0)),
            ],
            out_specs=pl.BlockSpec((BB, 2 * Z, 2 * Z), lambda i, c: (i, 0, 0)),
        ),
        out_shape=jax.ShapeDtypeStruct((b, 2 * Z, 2 * Z), jnp.float32),
    )(combo, cov_t)


def _sc_mu_gather(b):
    info = plsc.get_sparse_core_info()
    nc, ns = info.num_cores, info.num_subcores
    nw = nc * ns
    bpw = b // nw
    mesh = plsc.VectorSubcoreMesh(core_axis_name="c", subcore_axis_name="s")

    @functools.partial(
        pl.kernel,
        mesh=mesh,
        out_type=jax.ShapeDtypeStruct((b, 2 * Z), jnp.float32),
        scratch_types=[
            pltpu.VMEM((bpw,), jnp.int32),
            pltpu.VMEM((bpw, 2 * Z), jnp.float32),
            pltpu.SemaphoreType.DMA,
        ],
    )
    def k(mu_t, combo, mu_out, combo_v, rows_v, sem):
        wid = lax.axis_index("s") * nc + lax.axis_index("c")
        base = wid * bpw
        pltpu.sync_copy(combo.at[pl.ds(base, bpw)], combo_v)
        pltpu.async_copy(mu_t.at[combo_v], rows_v, sem).wait()
        pltpu.sync_copy(rows_v, mu_out.at[pl.ds(base, bpw)])

    return k


def kernel(y, e, mu_causal, low_rank_causal, diag_causal,
           mu_spurious, low_rank_spurious, diag_spurious):
    combo = (y.astype(jnp.int32) * 2 + e.astype(jnp.int32))
    cov_t, mu_t = _build_tables(mu_causal, low_rank_causal, diag_causal,
                                mu_spurious, low_rank_spurious, diag_spurious)
    b = y.shape[0]
    mu = _sc_mu_gather(b)(mu_t, combo)
    cov = _assemble_cov(combo, cov_t, b)
    return (mu, cov)
